# baseline (speedup 1.0000x reference)
import dataclasses
import functools
import math

import jax
import jax.numpy as jnp
from jax import lax
from jax.experimental import pallas as pl
from jax.experimental.pallas import tpu as pltpu
from jax.experimental.pallas import tpu_sc as plsc

N = 100000
IN_FEATURES = 128
HIDDEN = 64
NUM_SEGMENTS = 512
SHIFT = float(math.log(2.0))

BLOCK = 25088
N_PAD = 100352

NUM_SUBCORES = 16
LANES = 16
CHUNK = N_PAD // NUM_SUBCORES
ROWS_PER_SUBCORE = NUM_SEGMENTS // NUM_SUBCORES


LOG2E = float(1.0 / math.log(2.0))
LN2 = float(math.log(2.0))


def _mlp_body(scaler_ref, w1_ref, b1_ref, w2_ref, b2_ref, q_ref):
    x = scaler_ref[...].astype(jnp.bfloat16)
    w1 = w1_ref[...].astype(jnp.bfloat16)
    h = jnp.dot(x, w1, preferred_element_type=jnp.float32) + b1_ref[...]
    t = jnp.exp2(jnp.abs(h) * (-LOG2E))
    h = jnp.maximum(h, 0.0) + LN2 * (jnp.log2(1.0 + t) - 1.0)
    q_ref[...] = jnp.dot(h, w2_ref[...], preferred_element_type=jnp.float32) + b2_ref[...]


def _mlp_call(scaler, W1, b1, W2, b2):
    q = pl.pallas_call(
        _mlp_body,
        grid=(N_PAD // BLOCK,),
        in_specs=[
            pl.BlockSpec((BLOCK, IN_FEATURES), lambda i: (i, 0)),
            pl.BlockSpec((IN_FEATURES, HIDDEN), lambda i: (0, 0)),
            pl.BlockSpec((1, HIDDEN), lambda i: (0, 0)),
            pl.BlockSpec((HIDDEN, 1), lambda i: (0, 0)),
            pl.BlockSpec((1, 1), lambda i: (0, 0)),
        ],
        out_specs=pl.BlockSpec((BLOCK, 1), lambda i: (i, 0)),
        out_shape=jax.ShapeDtypeStruct((N_PAD, 1), jnp.float32),
    )(scaler, W1, b1.reshape(1, HIDDEN), W2, b2.reshape(1, 1))
    return q.reshape(N_PAD)


_SC_COMPILER_PARAMS = pltpu.CompilerParams()
if "needs_layout_passes" in pltpu.CompilerParams.__dataclass_fields__:
    _SC_COMPILER_PARAMS = dataclasses.replace(
        _SC_COMPILER_PARAMS, needs_layout_passes=False
    )


@functools.partial(
    pl.kernel,
    compiler_params=_SC_COMPILER_PARAMS,
    out_type=jax.ShapeDtypeStruct((NUM_SEGMENTS,), jnp.float32),
    mesh=plsc.VectorSubcoreMesh(core_axis_name="c", subcore_axis_name="s"),
    scratch_types=[
        pltpu.VMEM((CHUNK,), jnp.float32),
        pltpu.VMEM((CHUNK,), jnp.float32),
        pltpu.VMEM((CHUNK,), jnp.float32),
        pltpu.VMEM((CHUNK,), jnp.float32),
        pltpu.VMEM((CHUNK,), jnp.int32),
        pltpu.VMEM((NUM_SEGMENTS, LANES), jnp.float32),
        pltpu.VMEM((NUM_SEGMENTS,), jnp.float32),
        pltpu.VMEM((NUM_SEGMENTS,), jnp.int32),
        pltpu.VMEM((NUM_SEGMENTS,), jnp.float32),
        pltpu.VMEM((ROWS_PER_SUBCORE,), jnp.float32),
        pltpu.VMEM_SHARED((NUM_SEGMENTS,), jnp.float32),
        pltpu.SemaphoreType.DMA,
    ],
)
def _segsum(
    q_hbm, pos_hbm, id_hbm, out_hbm,
    q_loc, x_loc, y_loc, z_loc, id_loc,
    acc, red, iota512, zbuf, outbuf, shared, sem,
):
    cid = lax.axis_index("c")
    sid = lax.axis_index("s")

    @pl.when(cid == 0)
    def _():
        base = sid * CHUNK
        c1 = pltpu.async_copy(q_hbm.at[pl.ds(base, CHUNK)], q_loc, sem)
        c2 = pltpu.async_copy(pos_hbm.at[pl.ds(base, CHUNK)], x_loc, sem)
        c3 = pltpu.async_copy(pos_hbm.at[pl.ds(N_PAD + base, CHUNK)], y_loc, sem)
        c4 = pltpu.async_copy(pos_hbm.at[pl.ds(2 * N_PAD + base, CHUNK)], z_loc, sem)
        c5 = pltpu.async_copy(id_hbm.at[pl.ds(base, CHUNK)], id_loc, sem)

        zeros16 = jnp.zeros((LANES,), jnp.float32)
        lane = lax.iota(jnp.int32, LANES)

        @pl.loop(0, NUM_SEGMENTS)
        def _(r):
            acc[r] = zeros16

        @pl.loop(0, NUM_SEGMENTS, step=LANES)
        def _(r):
            iota512[pl.ds(r, LANES)] = lane + r

        @pl.when(sid == 0)
        def _():
            @pl.loop(0, NUM_SEGMENTS, step=LANES)
            def _(r):
                zbuf[pl.ds(r, LANES)] = zeros16

            pltpu.sync_copy(zbuf, shared)

        c1.wait()
        c2.wait()
        c3.wait()
        c4.wait()
        c5.wait()

        @pl.loop(0, CHUNK, step=LANES)
        def _(i):
            ids = id_loc[pl.ds(i, LANES)]
            xs = x_loc[pl.ds(i, LANES)]
            ys = y_loc[pl.ds(i, LANES)]
            zs = z_loc[pl.ds(i, LANES)]
            r2 = xs * xs + ys * ys + zs * zs
            vals = jnp.where(r2 > 0.0, q_loc[pl.ds(i, LANES)] * r2, 0.0)
            plsc.addupdate_scatter(acc, [ids, lane], vals)

        @pl.loop(0, NUM_SEGMENTS, step=LANES)
        def _(r):
            rows = r + lane
            tot = zeros16
            for l in range(LANES):
                col = jnp.full((LANES,), l, jnp.int32)
                tot = tot + plsc.load_gather(acc, [rows, col])
            red[pl.ds(r, LANES)] = tot

        plsc.subcore_barrier()
        pltpu.sync_copy(red, shared.at[iota512], add=True)
        plsc.subcore_barrier()

        rowbase = sid * ROWS_PER_SUBCORE
        pltpu.sync_copy(shared.at[pl.ds(rowbase, ROWS_PER_SUBCORE)], outbuf)
        pltpu.sync_copy(outbuf, out_hbm.at[pl.ds(rowbase, ROWS_PER_SUBCORE)])


def kernel(pos, scaler, vector, W1, b1, W2, b2, batch_index):
    del vector
    ids = jnp.pad(batch_index.astype(jnp.int32), (0, N_PAD - N))
    pos_t = jnp.pad(pos.T, ((0, 0), (0, N_PAD - N))).reshape(3 * N_PAD)
    q = _mlp_call(scaler, W1, b1, W2, b2)
    out = _segsum(q, pos_t, ids)
    return out.reshape(NUM_SEGMENTS, 1)


# speedup vs baseline: 1.0130x; 1.0130x over previous
import dataclasses
import functools
import math

import jax
import jax.numpy as jnp
from jax import lax
from jax.experimental import pallas as pl
from jax.experimental.pallas import tpu as pltpu
from jax.experimental.pallas import tpu_sc as plsc

N = 100000
IN_FEATURES = 128
HIDDEN = 64
NUM_SEGMENTS = 512
SHIFT = float(math.log(2.0))

BLOCK = 12544
N_PAD = 100352

NUM_SUBCORES = 16
LANES = 16
CHUNK = N_PAD // NUM_SUBCORES
ROWS_PER_SUBCORE = NUM_SEGMENTS // NUM_SUBCORES


LOG2E = float(1.0 / math.log(2.0))
LN2 = float(math.log(2.0))


def _mlp_body(scaler_ref, w1_ref, b1_ref, w2_ref, b2_ref, q_ref):
    x = scaler_ref[...]
    h = jnp.dot(x, w1_ref[...], preferred_element_type=jnp.float32) + b1_ref[...]
    t = jnp.exp2(jnp.abs(h) * (-LOG2E))
    h = jnp.maximum(h, 0.0) + LN2 * (jnp.log2(1.0 + t) - 1.0)
    q_ref[...] = jnp.dot(h, w2_ref[...], preferred_element_type=jnp.float32) + b2_ref[...]


def _mlp_call(scaler, W1, b1, W2, b2):
    q = pl.pallas_call(
        _mlp_body,
        grid=(N_PAD // BLOCK,),
        in_specs=[
            pl.BlockSpec((BLOCK, IN_FEATURES), lambda i: (i, 0)),
            pl.BlockSpec((IN_FEATURES, HIDDEN), lambda i: (0, 0)),
            pl.BlockSpec((1, HIDDEN), lambda i: (0, 0)),
            pl.BlockSpec((HIDDEN, 1), lambda i: (0, 0)),
            pl.BlockSpec((1, 1), lambda i: (0, 0)),
        ],
        out_specs=pl.BlockSpec((BLOCK, 1), lambda i: (i, 0)),
        out_shape=jax.ShapeDtypeStruct((N_PAD, 1), jnp.float32),
    )(scaler, W1, b1.reshape(1, HIDDEN), W2, b2.reshape(1, 1))
    return q.reshape(N_PAD)


_SC_COMPILER_PARAMS = pltpu.CompilerParams()
if "needs_layout_passes" in pltpu.CompilerParams.__dataclass_fields__:
    _SC_COMPILER_PARAMS = dataclasses.replace(
        _SC_COMPILER_PARAMS, needs_layout_passes=False
    )


@functools.partial(
    pl.kernel,
    compiler_params=_SC_COMPILER_PARAMS,
    out_type=jax.ShapeDtypeStruct((NUM_SEGMENTS,), jnp.float32),
    mesh=plsc.VectorSubcoreMesh(core_axis_name="c", subcore_axis_name="s"),
    scratch_types=[
        pltpu.VMEM((CHUNK,), jnp.float32),
        pltpu.VMEM((CHUNK,), jnp.float32),
        pltpu.VMEM((CHUNK,), jnp.float32),
        pltpu.VMEM((CHUNK,), jnp.float32),
        pltpu.VMEM((CHUNK,), jnp.int32),
        pltpu.VMEM((NUM_SEGMENTS, LANES), jnp.float32),
        pltpu.VMEM((NUM_SEGMENTS,), jnp.float32),
        pltpu.VMEM((NUM_SEGMENTS,), jnp.int32),
        pltpu.VMEM((NUM_SEGMENTS,), jnp.float32),
        pltpu.VMEM((ROWS_PER_SUBCORE,), jnp.float32),
        pltpu.VMEM_SHARED((NUM_SEGMENTS,), jnp.float32),
        pltpu.SemaphoreType.DMA,
    ],
)
def _segsum(
    q_hbm, pos_hbm, id_hbm, out_hbm,
    q_loc, x_loc, y_loc, z_loc, id_loc,
    acc, red, iota512, zbuf, outbuf, shared, sem,
):
    cid = lax.axis_index("c")
    sid = lax.axis_index("s")

    @pl.when(cid == 0)
    def _():
        base = sid * CHUNK
        c1 = pltpu.async_copy(q_hbm.at[pl.ds(base, CHUNK)], q_loc, sem)
        c2 = pltpu.async_copy(pos_hbm.at[pl.ds(base, CHUNK)], x_loc, sem)
        c3 = pltpu.async_copy(pos_hbm.at[pl.ds(N_PAD + base, CHUNK)], y_loc, sem)
        c4 = pltpu.async_copy(pos_hbm.at[pl.ds(2 * N_PAD + base, CHUNK)], z_loc, sem)
        c5 = pltpu.async_copy(id_hbm.at[pl.ds(base, CHUNK)], id_loc, sem)

        zeros16 = jnp.zeros((LANES,), jnp.float32)
        lane = lax.iota(jnp.int32, LANES)

        @pl.loop(0, NUM_SEGMENTS)
        def _(r):
            acc[r] = zeros16

        @pl.loop(0, NUM_SEGMENTS, step=LANES)
        def _(r):
            iota512[pl.ds(r, LANES)] = lane + r

        @pl.when(sid == 0)
        def _():
            @pl.loop(0, NUM_SEGMENTS, step=LANES)
            def _(r):
                zbuf[pl.ds(r, LANES)] = zeros16

            pltpu.sync_copy(zbuf, shared)

        c1.wait()
        c2.wait()
        c3.wait()
        c4.wait()
        c5.wait()

        @pl.loop(0, CHUNK, step=LANES)
        def _(i):
            ids = id_loc[pl.ds(i, LANES)]
            xs = x_loc[pl.ds(i, LANES)]
            ys = y_loc[pl.ds(i, LANES)]
            zs = z_loc[pl.ds(i, LANES)]
            r2 = xs * xs + ys * ys + zs * zs
            vals = jnp.where(r2 > 0.0, q_loc[pl.ds(i, LANES)] * r2, 0.0)
            plsc.addupdate_scatter(acc, [ids, lane], vals)

        @pl.loop(0, NUM_SEGMENTS, step=LANES)
        def _(r):
            rows = r + lane
            tot = zeros16
            for l in range(LANES):
                col = jnp.full((LANES,), l, jnp.int32)
                tot = tot + plsc.load_gather(acc, [rows, col])
            red[pl.ds(r, LANES)] = tot

        plsc.subcore_barrier()
        pltpu.sync_copy(red, shared.at[iota512], add=True)
        plsc.subcore_barrier()

        rowbase = sid * ROWS_PER_SUBCORE
        pltpu.sync_copy(shared.at[pl.ds(rowbase, ROWS_PER_SUBCORE)], outbuf)
        pltpu.sync_copy(outbuf, out_hbm.at[pl.ds(rowbase, ROWS_PER_SUBCORE)])


def kernel(pos, scaler, vector, W1, b1, W2, b2, batch_index):
    del vector
    ids = jnp.pad(batch_index.astype(jnp.int32), (0, N_PAD - N))
    pos_t = jnp.pad(pos.T, ((0, 0), (0, N_PAD - N))).reshape(3 * N_PAD)
    q = _mlp_call(scaler, W1, b1, W2, b2)
    out = _segsum(q, pos_t, ids)
    return out.reshape(NUM_SEGMENTS, 1)
